# baseline (speedup 1.0000x reference)
import jax
import jax.numpy as jnp
from jax import lax
from jax.experimental import pallas as pl
from jax.experimental.pallas import tpu as pltpu

_NE = 1024
_D = 64
_N_BLK = 4096


def _vq_body(z_ref, wm2_ref, w2_ref, wt_ref, zq_ref, idx_ref, sse_ref):
    zb = z_ref[0]
    wm2 = wm2_ref[...]
    w2 = w2_ref[...]
    wt = wt_ref[...]
    mT = lax.dot_general(wm2, zb, (((1,), (0,)), ((), ())),
                         preferred_element_type=jnp.float32)
    z2 = jnp.sum(zb * zb, axis=0)[None, :]
    d = (z2 + mT) + w2
    g = jnp.argmin(d, axis=0)
    iota = lax.broadcasted_iota(jnp.int32, (_NE, _N_BLK), 0)
    oh = jnp.where(iota == g[None, :], 1.0, 0.0)
    zq = lax.dot_general(wt, oh, (((1,), (0,)), ((), ())),
                         preferred_element_type=jnp.float32)
    zq_ref[0] = zb + (zq - zb)
    o = g & 7
    r = jnp.where(o == 0, 0, jnp.where(o == 1, 7, jnp.where(
        o == 2, 3, jnp.where(o == 3, 5, jnp.where(
            o == 4, 1, jnp.where(o == 5, 6, jnp.where(o == 6, 2, 4)))))))
    idx = (r << 7) | (g >> 3)
    idx_ref[...] = idx.reshape(1, 1, 1, _N_BLK)
    diff = zq - zb
    p = jnp.sum(diff * diff)
    first = (pl.program_id(0) == 0) & (pl.program_id(1) == 0)

    @pl.when(first)
    def _():
        sse_ref[0, 0] = 0.0

    sse_ref[0, 0] = sse_ref[0, 0] + p


def kernel(z, W):
    B, C, T, H, Wd = z.shape
    S = T * H * Wd
    z3 = z.reshape(B, C, S)
    rank = jnp.array([0, 7, 3, 5, 1, 6, 2, 4], jnp.int32)
    p = jnp.arange(_NE, dtype=jnp.int32)
    cmap = rank[p & 7] * (_NE // 8) + (p >> 3)
    w2_full = jnp.sum(W ** 2, axis=1)
    Wp = W[cmap]
    WT = Wp.T
    Wm2 = Wp * (-2.0)
    w2 = w2_full[cmap][:, None]
    nb = S // _N_BLK
    zq3, idx4, sse = pl.pallas_call(
        _vq_body,
        grid=(B, nb),
        in_specs=[
            pl.BlockSpec((1, C, _N_BLK), lambda b, n: (b, 0, n)),
            pl.BlockSpec((_NE, _D), lambda b, n: (0, 0)),
            pl.BlockSpec((_NE, 1), lambda b, n: (0, 0)),
            pl.BlockSpec((_D, _NE), lambda b, n: (0, 0)),
        ],
        out_specs=[
            pl.BlockSpec((1, C, _N_BLK), lambda b, n: (b, 0, n)),
            pl.BlockSpec((1, 1, 1, _N_BLK), lambda b, n: (b, n, 0, 0)),
            pl.BlockSpec(memory_space=pltpu.SMEM),
        ],
        out_shape=[
            jax.ShapeDtypeStruct((B, C, S), jnp.float32),
            jax.ShapeDtypeStruct((B, nb, 1, _N_BLK), jnp.int32),
            jax.ShapeDtypeStruct((1, 1), jnp.float32),
        ],
    )(z3, Wm2, w2, WT)
    zq_st = zq3.reshape(B, C, T, H, Wd)
    indices = idx4.reshape(B, T, H, Wd)
    mean = sse[0, 0] / (B * C * S)
    vq_loss = mean + 0.25 * mean
    return zq_st, vq_loss, indices


# speedup vs baseline: 1.1217x; 1.1217x over previous
import jax
import jax.numpy as jnp
from jax import lax
from jax.experimental import pallas as pl
from jax.experimental.pallas import tpu as pltpu

_NE = 1024
_D = 64
_N_BLK = 4096


def _vq_body(z_ref, wp_ref, zq_ref, idx_ref, sse_ref):
    zb = z_ref[0]
    wp = wp_ref[...]
    wm2 = wp * (-2.0)
    w2 = jnp.sum(wp * wp, axis=1)[:, None]
    mT = lax.dot_general(wm2, zb, (((1,), (0,)), ((), ())),
                         preferred_element_type=jnp.float32)
    z2 = jnp.sum(zb * zb, axis=0)[None, :]
    d = (z2 + mT) + w2
    g = jnp.argmin(d, axis=0)
    iota = lax.broadcasted_iota(jnp.int32, (_NE, _N_BLK), 0)
    oh = jnp.where(iota == g[None, :], 1.0, 0.0)
    zq = lax.dot_general(wp, oh, (((0,), (0,)), ((), ())),
                         preferred_element_type=jnp.float32)
    zq_ref[0] = zb + (zq - zb)
    o = g & 7
    r = jnp.where(o == 0, 0, jnp.where(o == 1, 7, jnp.where(
        o == 2, 3, jnp.where(o == 3, 5, jnp.where(
            o == 4, 1, jnp.where(o == 5, 6, jnp.where(o == 6, 2, 4)))))))
    idx = (r << 7) | (g >> 3)
    idx_ref[...] = idx.reshape(1, 1, 1, _N_BLK)
    diff = zq - zb
    p = jnp.sum(diff * diff)
    first = (pl.program_id(0) == 0) & (pl.program_id(1) == 0)

    @pl.when(first)
    def _():
        sse_ref[0, 0] = 0.0

    sse_ref[0, 0] = sse_ref[0, 0] + p


def kernel(z, W):
    B, C, T, H, Wd = z.shape
    S = T * H * Wd
    z3 = z.reshape(B, C, S)
    rank = jnp.array([0, 7, 3, 5, 1, 6, 2, 4], jnp.int32)
    p = jnp.arange(_NE, dtype=jnp.int32)
    cmap = rank[p & 7] * (_NE // 8) + (p >> 3)
    Wp = W[cmap]
    nb = S // _N_BLK
    zq3, idx4, sse = pl.pallas_call(
        _vq_body,
        grid=(B, nb),
        in_specs=[
            pl.BlockSpec((1, C, _N_BLK), lambda b, n: (b, 0, n)),
            pl.BlockSpec((_NE, _D), lambda b, n: (0, 0)),
        ],
        out_specs=[
            pl.BlockSpec((1, C, _N_BLK), lambda b, n: (b, 0, n)),
            pl.BlockSpec((1, 1, 1, _N_BLK), lambda b, n: (b, n, 0, 0)),
            pl.BlockSpec(memory_space=pltpu.SMEM),
        ],
        out_shape=[
            jax.ShapeDtypeStruct((B, C, S), jnp.float32),
            jax.ShapeDtypeStruct((B, nb, 1, _N_BLK), jnp.int32),
            jax.ShapeDtypeStruct((1, 1), jnp.float32),
        ],
    )(z3, Wp)
    zq_st = zq3.reshape(B, C, T, H, Wd)
    indices = idx4.reshape(B, T, H, Wd)
    mean = sse[0, 0] / (B * C * S)
    vq_loss = mean + 0.25 * mean
    return zq_st, vq_loss, indices
